# baseline (speedup 1.0000x reference)
import numpy as np

import jax
import jax.numpy as jnp
from jax.experimental import pallas as pl
from jax.experimental.pallas import tpu as pltpu

IMG_W = 16
IMG_PIX = 256
CNN_CH = 8
L_UTT = 512
U_DLG = 4
ADAPT = 4
CHUNK = 512
D_ROWS = L_UTT * U_DLG
K_STEPS = D_ROWS // CHUNK

_PMAT = np.zeros((ADAPT * U_DLG, D_ROWS), np.float32)
for _s in range(ADAPT):
    for _u in range(U_DLG):
        _b = _u * L_UTT + _s * (L_UTT // ADAPT)
        _PMAT[_s * U_DLG + _u, _b:_b + L_UTT // ADAPT] = 1.0 / (L_UTT // ADAPT)

_A2 = np.zeros((IMG_PIX, IMG_PIX, 9), np.float32)
for _t, (_di, _dj) in enumerate([(di, dj) for di in (-1, 0, 1)
                                 for dj in (-1, 0, 1)]):
    for _p in range(IMG_PIX):
        _q = _p + _di * IMG_W + _dj
        if not (0 <= _q < IMG_PIX):
            continue
        _c0 = _p % IMG_W
        if (_dj == -1 and _c0 == 0) or (_dj == 1 and _c0 == IMG_W - 1):
            continue
        _A2[_q, _p, _t] = 1.0


def _fused_kernel(x_ref, pe_ref,
                  w2_ref, bias_pat_ref, gfc_ref, fc_b_ref,
                  cap_w_ref, cap_b_ref,
                  gate_w_ref, gate_b_ref, tcn_w_ref, tcn_b_ref,
                  gws_ref, g_b_ref, cls_w_ref, cls_b_ref, pmat_ref,
                  o_ref, feats_ref, stats_ref):
    d = pl.program_id(0)
    k = pl.program_id(1)
    step = d * K_STEPS + k

    @pl.when(step == 0)
    def _stats():
        pe_all = pe_ref[...]
        n = pe_all.size
        s1 = jnp.sum(pe_all) / n
        s2 = jnp.sum(pe_all * pe_all) / n
        stats_ref[0, 0] = s1
        stats_ref[0, 1] = jax.lax.rsqrt(s2 - s1 * s1 + 1e-5)

    x = x_ref[...].reshape(CHUNK, IMG_PIX)

    big = jnp.dot(x, w2_ref[...], preferred_element_type=jnp.float32)
    big = jnp.maximum(big + bias_pat_ref[...], 0.0)
    cnn_feats = (jnp.dot(big, gfc_ref[...], preferred_element_type=jnp.float32)
                 + fc_b_ref[...])
    cap_feats = jnp.tanh(
        jnp.dot(x, cap_w_ref[...], preferred_element_type=jnp.float32)
        + cap_b_ref[...])
    pe = pe_ref[pl.ds(step * CHUNK, CHUNK), :]
    pe_n = (pe - stats_ref[0, 0]) * stats_ref[0, 1]

    cc = jnp.concatenate([cnn_feats, cap_feats, pe_n], axis=-1)
    gate = jax.nn.sigmoid(
        jnp.dot(cc, gate_w_ref[...], preferred_element_type=jnp.float32)
        + gate_b_ref[...])
    feats_ref[pl.ds(k * CHUNK, CHUNK), :] = gate * cc

    @pl.when(k == K_STEPS - 1)
    def _dialog():
        X = feats_ref[...]
        t0 = jnp.dot(X, tcn_w_ref[0], preferred_element_type=jnp.float32)
        t1 = jnp.dot(X, tcn_w_ref[1], preferred_element_type=jnp.float32)
        t2 = jnp.dot(X, tcn_w_ref[2], preferred_element_type=jnp.float32)

        rows = jax.lax.broadcasted_iota(jnp.int32, (D_ROWS, 1), 0) % L_UTT
        z = jnp.zeros((1, 128), jnp.float32)
        t1s = jnp.where(rows >= 1,
                        jnp.concatenate([z, t1[:-1, :]], axis=0), 0.0)
        t0s = jnp.where(rows >= 2,
                        jnp.concatenate([z, z, t0[:-2, :]], axis=0), 0.0)
        y = jnp.maximum(t2 + t1s + t0s + tcn_b_ref[...], 0.0) + X

        pooled = jnp.dot(pmat_ref[...], y,
                         preferred_element_type=jnp.float32)
        xw = jnp.dot(pooled[0:U_DLG], gws_ref[0],
                     preferred_element_type=jnp.float32)
        for s in range(1, ADAPT):
            xw = xw + jnp.dot(pooled[s * U_DLG:(s + 1) * U_DLG], gws_ref[s],
                              preferred_element_type=jnp.float32)

        xw1 = xw[:, 0:128]
        xw2 = xw[:, 128:256]
        xw3 = xw[:, 256:384]
        h1 = jnp.mean(xw1, axis=0, keepdims=True)
        h2 = 0.5 * (xw2 + jnp.concatenate([xw2[2:], xw2[:2]], axis=0))
        h = jnp.maximum(h1 + h2 + xw3 + g_b_ref[...], 0.0)
        out = (jnp.dot(h, cls_w_ref[...], preferred_element_type=jnp.float32)
               + cls_b_ref[...])
        o_ref[...] = out.reshape(1, U_DLG, 128)


@jax.jit
def _forward(spectrograms, pretrain_embedding, cnn_wb, fc_w, fc_b, cap_w,
             cap_b, gate_w, gate_b, tcn_w, tcn_b, g_w_slab, g_b, cls_w, cls_b):
    N = spectrograms.shape[0]
    n_dlg = N // D_ROWS
    x = spectrograms.astype(jnp.float32)
    pe = pretrain_embedding.astype(jnp.float32)
    pmat = jnp.asarray(_PMAT)

    qi = jax.lax.broadcasted_iota(jnp.int32, (IMG_PIX, IMG_PIX * CNN_CH), 0)
    colx = jax.lax.broadcasted_iota(jnp.int32, (IMG_PIX, IMG_PIX * CNN_CH), 1)
    p = colx // CNN_CH
    pc = p % IMG_W
    w2 = jnp.zeros((IMG_PIX, IMG_PIX * CNN_CH), jnp.float32)
    t = 0
    for di in (-1, 0, 1):
        for dj in (-1, 0, 1):
            valid = qi == p + di * IMG_W + dj
            if dj == -1:
                valid = valid & (pc != 0)
            elif dj == 1:
                valid = valid & (pc != IMG_W - 1)
            wpat = jnp.tile(cnn_wb[t], IMG_PIX)[None, :]
            w2 = w2 + jnp.where(valid, wpat, 0.0)
            t += 1
    bias_pat = jnp.tile(cnn_wb[9], IMG_PIX).reshape(1, IMG_PIX * CNN_CH)
    gfc = jnp.tile(fc_w * (1.0 / IMG_PIX), (IMG_PIX, 1))

    out = pl.pallas_call(
        _fused_kernel,
        out_shape=jax.ShapeDtypeStruct((n_dlg, U_DLG, 128), jnp.float32),
        grid=(n_dlg, K_STEPS),
        in_specs=[
                pl.BlockSpec((CHUNK, 1, 16, 16),
                             lambda d, k: (d * K_STEPS + k, 0, 0, 0)),
                pl.BlockSpec((N, 64), lambda d, k: (0, 0)),

                pl.BlockSpec((IMG_PIX, IMG_PIX * CNN_CH),
                             lambda d, k: (0, 0)),
                pl.BlockSpec((1, IMG_PIX * CNN_CH), lambda d, k: (0, 0)),
                pl.BlockSpec((IMG_PIX * CNN_CH, 32), lambda d, k: (0, 0)),
                pl.BlockSpec((1, 32), lambda d, k: (0, 0)),
                pl.BlockSpec((IMG_PIX, 32), lambda d, k: (0, 0)),
                pl.BlockSpec((1, 32), lambda d, k: (0, 0)),
                pl.BlockSpec((128, 128), lambda d, k: (0, 0)),
                pl.BlockSpec((1, 128), lambda d, k: (0, 0)),
                pl.BlockSpec((3, 128, 128), lambda d, k: (0, 0, 0)),
                pl.BlockSpec((1, 128), lambda d, k: (0, 0)),
                pl.BlockSpec((ADAPT, 128, 384), lambda d, k: (0, 0, 0)),
                pl.BlockSpec((1, 128), lambda d, k: (0, 0)),
                pl.BlockSpec((128, 128), lambda d, k: (0, 0)),
                pl.BlockSpec((1, 128), lambda d, k: (0, 0)),
                pl.BlockSpec((ADAPT * U_DLG, D_ROWS), lambda d, k: (0, 0)),
        ],
        out_specs=pl.BlockSpec((1, U_DLG, 128), lambda d, k: (d, 0, 0)),
        scratch_shapes=[pltpu.VMEM((D_ROWS, 128), jnp.float32),
                        pltpu.SMEM((1, 2), jnp.float32)],
        compiler_params=pltpu.CompilerParams(
            dimension_semantics=("arbitrary", "arbitrary")),
        cost_estimate=pl.CostEstimate(
            flops=int(N * (CNN_CH * 9 * IMG_PIX * 2 + CNN_CH * 32 * 2
                           + IMG_PIX * 32 * 2 + 128 * 128 * 2
                           + 3 * 128 * 128 * 2 + ADAPT * 3 * 128 * 2)),
            transcendentals=int(N * (32 + 128)),
            bytes_accessed=int(4 * (N * IMG_PIX + N * 64 + n_dlg * U_DLG * 128))),
    )(x, pe, w2, bias_pat, gfc, fc_b, cap_w, cap_b, gate_w, gate_b,
      tcn_w, tcn_b, g_w_slab, g_b, cls_w, cls_b, pmat)
    return out.reshape(n_dlg * U_DLG, 128)[:, :4]


def kernel(spectrograms, pretrain_embedding, cnn_wb, fc_w, fc_b, cap_w, cap_b,
           gate_w, gate_b, tcn_w, tcn_b, g_w_slab, g_b, cls_w, cls_b):
    return _forward(spectrograms, pretrain_embedding, cnn_wb, fc_w, fc_b,
                    cap_w, cap_b, gate_w, gate_b, tcn_w, tcn_b, g_w_slab,
                    g_b, cls_w, cls_b)


# speedup vs baseline: 2.3155x; 2.3155x over previous
import numpy as np

import jax
import jax.numpy as jnp
from jax.experimental import pallas as pl
from jax.experimental.pallas import tpu as pltpu

IMG_W = 16
IMG_PIX = 256
CNN_CH = 8
L_UTT = 512
U_DLG = 4
ADAPT = 4
D_ROWS = L_UTT * U_DLG
GAP_FOLD = 3
GAP_K = (IMG_PIX >> GAP_FOLD) * CNN_CH

_PMAT = np.zeros((ADAPT * U_DLG, D_ROWS), np.float32)
for _s in range(ADAPT):
    for _u in range(U_DLG):
        _b = _u * L_UTT + _s * (L_UTT // ADAPT)
        _PMAT[_s * U_DLG + _u, _b:_b + L_UTT // ADAPT] = 1.0 / (L_UTT // ADAPT)


def _fused_kernel(x_ref, pe_ref,
                  w2_ref, bias_pat_ref, gfc_ref, fc_b_ref,
                  cap_w_ref, cap_b_ref,
                  gate_w_ref, gate_b_ref, tcn_cat_ref, tcn_b_ref,
                  gws_ref, g_b_ref, cls_w_ref, cls_b_ref, pmat_ref,
                  o_ref, stats_ref):
    d = pl.program_id(0)

    @pl.when(d == 0)
    def _stats():
        pe_all = pe_ref[...]
        n = pe_all.size
        s1 = jnp.sum(pe_all) / n
        s2 = jnp.sum(pe_all * pe_all) / n
        stats_ref[0, 0] = s1
        stats_ref[0, 1] = jax.lax.rsqrt(s2 - s1 * s1 + 1e-5)

    x = x_ref[...]

    big = jnp.dot(x, w2_ref[...], preferred_element_type=jnp.float32)
    big = jnp.maximum(big + bias_pat_ref[...], 0.0)
    for _ in range(GAP_FOLD):
        half = big.shape[1] // 2
        big = big[:, :half] + big[:, half:]
    cnn_feats = (jnp.dot(big, gfc_ref[...], preferred_element_type=jnp.float32)
                 + fc_b_ref[...])

    cap_feats = jnp.tanh(
        jnp.dot(x, cap_w_ref[...], preferred_element_type=jnp.float32)
        + cap_b_ref[...])
    pe = pe_ref[pl.ds(d * D_ROWS, D_ROWS), :]
    pe_n = (pe - stats_ref[0, 0]) * stats_ref[0, 1]

    cc = jnp.concatenate([cnn_feats, cap_feats, pe_n], axis=-1)
    gate = jax.nn.sigmoid(
        jnp.dot(cc, gate_w_ref[...], preferred_element_type=jnp.float32)
        + gate_b_ref[...])
    X = gate * cc

    taps = jnp.dot(X, tcn_cat_ref[...], preferred_element_type=jnp.float32)
    t0 = taps[:, 0:128]
    t1 = taps[:, 128:256]
    t2 = taps[:, 256:384]

    rows = jax.lax.broadcasted_iota(jnp.int32, (D_ROWS, 1), 0) % L_UTT
    z = jnp.zeros((1, 128), jnp.float32)
    t1s = jnp.where(rows >= 1, jnp.concatenate([z, t1[:-1, :]], axis=0), 0.0)
    t0s = jnp.where(rows >= 2, jnp.concatenate([z, z, t0[:-2, :]], axis=0), 0.0)
    y = jnp.maximum(t2 + t1s + t0s + tcn_b_ref[...], 0.0) + X

    pooled = jnp.dot(pmat_ref[...], y,
                     preferred_element_type=jnp.float32)
    xw = jnp.dot(pooled[0:U_DLG], gws_ref[0],
                 preferred_element_type=jnp.float32)
    for s in range(1, ADAPT):
        xw = xw + jnp.dot(pooled[s * U_DLG:(s + 1) * U_DLG], gws_ref[s],
                          preferred_element_type=jnp.float32)

    xw1 = xw[:, 0:128]
    xw2 = xw[:, 128:256]
    xw3 = xw[:, 256:384]
    h1 = jnp.mean(xw1, axis=0, keepdims=True)
    h2 = 0.5 * (xw2 + jnp.concatenate([xw2[2:], xw2[:2]], axis=0))
    h = jnp.maximum(h1 + h2 + xw3 + g_b_ref[...], 0.0)
    out = (jnp.dot(h, cls_w_ref[...], preferred_element_type=jnp.float32)
           + cls_b_ref[...])
    o_ref[...] = out.reshape(1, U_DLG, 128)


@jax.jit
def _forward(spectrograms, pretrain_embedding, cnn_wb, fc_w, fc_b, cap_w,
             cap_b, gate_w, gate_b, tcn_w, tcn_b, g_w_slab, g_b, cls_w, cls_b):
    N = spectrograms.shape[0]
    n_dlg = N // D_ROWS
    x = spectrograms.astype(jnp.float32).reshape(N, IMG_PIX)
    pe = pretrain_embedding.astype(jnp.float32)
    pmat = jnp.asarray(_PMAT)

    qi = jax.lax.broadcasted_iota(jnp.int32, (IMG_PIX, IMG_PIX * CNN_CH), 0)
    colx = jax.lax.broadcasted_iota(jnp.int32, (IMG_PIX, IMG_PIX * CNN_CH), 1)
    p = colx // CNN_CH
    pc = p % IMG_W
    w2 = jnp.zeros((IMG_PIX, IMG_PIX * CNN_CH), jnp.float32)
    t = 0
    for di in (-1, 0, 1):
        for dj in (-1, 0, 1):
            valid = qi == p + di * IMG_W + dj
            if dj == -1:
                valid = valid & (pc != 0)
            elif dj == 1:
                valid = valid & (pc != IMG_W - 1)
            wpat = jnp.tile(cnn_wb[t], IMG_PIX)[None, :]
            w2 = w2 + jnp.where(valid, wpat, 0.0)
            t += 1
    bias_pat = jnp.tile(cnn_wb[9], IMG_PIX).reshape(1, IMG_PIX * CNN_CH)
    gfc = jnp.tile(fc_w * (1.0 / IMG_PIX), (GAP_K // CNN_CH, 1))
    tcn_cat = jnp.concatenate([tcn_w[0], tcn_w[1], tcn_w[2]], axis=1)

    out = pl.pallas_call(
        _fused_kernel,
        out_shape=jax.ShapeDtypeStruct((n_dlg, U_DLG, 128), jnp.float32),
        grid=(n_dlg,),
        in_specs=[
            pl.BlockSpec((D_ROWS, IMG_PIX), lambda d: (d, 0)),
            pl.BlockSpec((N, 64), lambda d: (0, 0)),
            pl.BlockSpec((IMG_PIX, IMG_PIX * CNN_CH), lambda d: (0, 0)),
            pl.BlockSpec((1, IMG_PIX * CNN_CH), lambda d: (0, 0)),
            pl.BlockSpec((GAP_K, 32), lambda d: (0, 0)),
            pl.BlockSpec((1, 32), lambda d: (0, 0)),
            pl.BlockSpec((IMG_PIX, 32), lambda d: (0, 0)),
            pl.BlockSpec((1, 32), lambda d: (0, 0)),
            pl.BlockSpec((128, 128), lambda d: (0, 0)),
            pl.BlockSpec((1, 128), lambda d: (0, 0)),
            pl.BlockSpec((128, 384), lambda d: (0, 0)),
            pl.BlockSpec((1, 128), lambda d: (0, 0)),
            pl.BlockSpec((ADAPT, 128, 384), lambda d: (0, 0, 0)),
            pl.BlockSpec((1, 128), lambda d: (0, 0)),
            pl.BlockSpec((128, 128), lambda d: (0, 0)),
            pl.BlockSpec((1, 128), lambda d: (0, 0)),
            pl.BlockSpec((ADAPT * U_DLG, D_ROWS), lambda d: (0, 0)),
        ],
        out_specs=pl.BlockSpec((1, U_DLG, 128), lambda d: (d, 0, 0)),
        scratch_shapes=[pltpu.SMEM((1, 2), jnp.float32)],
        compiler_params=pltpu.CompilerParams(
            dimension_semantics=("arbitrary",)),
        cost_estimate=pl.CostEstimate(
            flops=int(N * (IMG_PIX * IMG_PIX * CNN_CH * 2 + GAP_K * 32 * 2
                           + IMG_PIX * 32 * 2 + 128 * 128 * 2
                           + 128 * 384 * 2 + ADAPT * 3 * 128 * 2)),
            transcendentals=int(N * (32 + 128)),
            bytes_accessed=int(4 * (N * IMG_PIX + N * 64 + n_dlg * U_DLG * 128))),
    )(x, pe, w2, bias_pat, gfc, fc_b, cap_w, cap_b, gate_w, gate_b,
      tcn_cat, tcn_b, g_w_slab, g_b, cls_w, cls_b, pmat)
    return out.reshape(n_dlg * U_DLG, 128)[:, :4]


def kernel(spectrograms, pretrain_embedding, cnn_wb, fc_w, fc_b, cap_w, cap_b,
           gate_w, gate_b, tcn_w, tcn_b, g_w_slab, g_b, cls_w, cls_b):
    return _forward(spectrograms, pretrain_embedding, cnn_wb, fc_w, fc_b,
                    cap_w, cap_b, gate_w, gate_b, tcn_w, tcn_b, g_w_slab,
                    g_b, cls_w, cls_b)
